# Initial kernel scaffold; baseline (speedup 1.0000x reference)
import jax
import jax.numpy as jnp
from jax.experimental import pallas as pl


def kernel(inputs, state, W, U, b, W_out, b_out):
    raise NotImplementedError("write your pallas kernel here")


# speedup vs baseline: 1.2635x; 1.2635x over previous
import functools

import jax
import jax.numpy as jnp
from jax import lax
from jax.experimental import pallas as pl
from jax.experimental.pallas import tpu as pltpu
from jax.experimental.pallas import tpu_sc as plsc

B = 16384
D_IN = 65
UNITS = 48
STATES = 1000
SPAD = 1024
BLK = 512
NBLK = B // BLK


DP = 128
IDX_CH = 128


def _sc_gather_rows(table, idx3d):
    V, D = table.shape
    nw, nch, _ = idx3d.shape
    n = nw * nch * IDX_CH
    b_per_w = nch * IDX_CH
    mesh = plsc.VectorSubcoreMesh(core_axis_name="c", subcore_axis_name="s")

    @functools.partial(
        pl.kernel, mesh=mesh,
        out_type=jax.ShapeDtypeStruct((n, D), jnp.float32),
        scratch_types=[
            pltpu.VMEM((nch, IDX_CH), jnp.int32),
            pltpu.VMEM((b_per_w, D), jnp.float32),
            pltpu.SemaphoreType.DMA,
        ],
    )
    def k(table_hbm, idx_hbm, out_hbm, idx_v, rows_v, sem):
        wid = lax.axis_index("s") * 2 + lax.axis_index("c")
        base = wid * b_per_w
        pltpu.sync_copy(idx_hbm.at[wid], idx_v)
        cps = [
            pltpu.async_copy(table_hbm.at[idx_v.at[j]],
                             rows_v.at[pl.ds(j * IDX_CH, IDX_CH)], sem)
            for j in range(nch)
        ]
        for cp in cps:
            cp.wait()
        pltpu.sync_copy(rows_v, out_hbm.at[pl.ds(base, b_per_w)])

    return k(table, idx3d)


def _gru_body(x_ref, h_ref, state_ref, W_ref, U_ref, b_ref, Wo_ref, bo_ref,
              out_ref, ns_ref):
    i = pl.program_id(0)
    x = x_ref[...]
    h = h_ref[:, :UNITS]
    xp = jnp.dot(x, W_ref[...], preferred_element_type=jnp.float32) + b_ref[...]
    hp = jnp.dot(h, U_ref[...], preferred_element_type=jnp.float32)
    u = UNITS
    z = jax.nn.sigmoid(xp[:, :u] + hp[:, :u])
    r = jax.nn.sigmoid(xp[:, u:2 * u] + hp[:, u:2 * u])
    hh = jnp.tanh(xp[:, 2 * u:] + r * hp[:, 2 * u:])
    h_new = z * h + (1.0 - z) * hh
    out_ref[...] = jax.nn.sigmoid(
        jnp.dot(h_new, Wo_ref[...], preferred_element_type=jnp.float32)
        + bo_ref[...])

    ids = jnp.clip(x[:, 0:1].astype(jnp.int32), 0, STATES - 1)
    sid = lax.broadcasted_iota(jnp.int32, (BLK, SPAD), 1)
    rid = lax.broadcasted_iota(jnp.int32, (BLK, SPAD), 0) + 1
    eq = ids == sid
    eqi = jnp.where(eq, rid, 0)
    bmax = jnp.max(eqi, axis=0, keepdims=True)
    sel = jnp.where(eq & (eqi == bmax), 1.0, 0.0)
    upd = lax.dot_general(sel, h_new, (((0,), (0,)), ((), ())),
                          preferred_element_type=jnp.float32)
    hit = (bmax > 0).T

    @pl.when(i == 0)
    def _():
        ns_ref[...] = state_ref[...]

    ns_ref[...] = jnp.where(hit[:STATES], upd[:STATES], ns_ref[...])


def _tc_gru(inputs, h_prev, state, W, U, b, W_out, b_out):
    out, new_state = pl.pallas_call(
        _gru_body,
        grid=(NBLK,),
        in_specs=[
            pl.BlockSpec((BLK, D_IN), lambda i: (i, 0)),
            pl.BlockSpec((BLK, DP), lambda i: (i, 0)),
            pl.BlockSpec((STATES, UNITS), lambda i: (0, 0)),
            pl.BlockSpec((D_IN, 3 * UNITS), lambda i: (0, 0)),
            pl.BlockSpec((UNITS, 3 * UNITS), lambda i: (0, 0)),
            pl.BlockSpec((1, 3 * UNITS), lambda i: (0, 0)),
            pl.BlockSpec((UNITS, 1), lambda i: (0, 0)),
            pl.BlockSpec((1, 1), lambda i: (0, 0)),
        ],
        out_specs=[
            pl.BlockSpec((BLK, 1), lambda i: (i, 0)),
            pl.BlockSpec((STATES, UNITS), lambda i: (0, 0)),
        ],
        out_shape=[
            jax.ShapeDtypeStruct((B, 1), jnp.float32),
            jax.ShapeDtypeStruct((STATES, UNITS), jnp.float32),
        ],
    )(inputs, h_prev, state, W, U, b.reshape(1, -1), W_out,
      b_out.reshape(1, 1))
    return out, new_state


def kernel(inputs, state, W, U, b, W_out, b_out):
    ids = jnp.clip(inputs[:, 0].astype(jnp.int32), 0, state.shape[0] - 1)
    nw = 32
    state_pad = jnp.pad(state, ((0, 0), (0, DP - UNITS)))
    idx3d = ids.reshape(nw, B // (nw * IDX_CH), IDX_CH)
    h_prev = _sc_gather_rows(state_pad, idx3d)
    return _tc_gru(inputs, h_prev, state, W, U, b, W_out, b_out)


# speedup vs baseline: 1.3516x; 1.0697x over previous
import functools

import jax
import jax.numpy as jnp
from jax import lax
from jax.experimental import pallas as pl
from jax.experimental.pallas import tpu as pltpu
from jax.experimental.pallas import tpu_sc as plsc

B = 16384
D_IN = 65
UNITS = 48
STATES = 1000
SPAD = 1024
BLK = 512
NBLK = B // BLK


DP = 128
IDX_CH = 128


def _sc_gather_rows(table, idx3d):
    V, D = table.shape
    nw, nch, _ = idx3d.shape
    n = nw * nch * IDX_CH
    b_per_w = nch * IDX_CH
    mesh = plsc.VectorSubcoreMesh(core_axis_name="c", subcore_axis_name="s")

    @functools.partial(
        pl.kernel, mesh=mesh,
        out_type=jax.ShapeDtypeStruct((n, D), jnp.float32),
        scratch_types=[
            pltpu.VMEM((nch, IDX_CH), jnp.int32),
            pltpu.VMEM((b_per_w, D), jnp.float32),
            pltpu.SemaphoreType.DMA,
        ],
    )
    def k(table_hbm, idx_hbm, out_hbm, idx_v, rows_v, sem):
        wid = lax.axis_index("s") * 2 + lax.axis_index("c")
        base = wid * b_per_w
        pltpu.sync_copy(idx_hbm.at[wid], idx_v)
        cps = [
            pltpu.async_copy(table_hbm.at[idx_v.at[j]],
                             rows_v.at[pl.ds(j * IDX_CH, IDX_CH)], sem)
            for j in range(nch)
        ]
        for cp in cps:
            cp.wait()
        pltpu.sync_copy(rows_v, out_hbm.at[pl.ds(base, b_per_w)])

    return k(table, idx3d)


def _gru_body(x_ref, h_ref, state_ref, W_ref, U_ref, b_ref, Wo_ref, bo_ref,
              out_ref, ns_ref):
    i = pl.program_id(0)
    x = x_ref[...]
    h = h_ref[:, :UNITS]
    xp = jnp.dot(x, W_ref[...], preferred_element_type=jnp.float32) + b_ref[...]
    hp = jnp.dot(h, U_ref[...], preferred_element_type=jnp.float32)
    u = UNITS
    z = jax.nn.sigmoid(xp[:, :u] + hp[:, :u])
    r = jax.nn.sigmoid(xp[:, u:2 * u] + hp[:, u:2 * u])
    hh = jnp.tanh(xp[:, 2 * u:] + r * hp[:, 2 * u:])
    h_new = z * h + (1.0 - z) * hh
    out_ref[...] = jax.nn.sigmoid(
        jnp.dot(h_new, Wo_ref[...], preferred_element_type=jnp.float32)
        + bo_ref[...])

    ids = x[:, 0:1].astype(jnp.int32)
    sid = lax.broadcasted_iota(jnp.int32, (BLK, SPAD), 1)
    rid = lax.broadcasted_iota(jnp.int32, (BLK, SPAD), 0)
    m = jnp.where(ids == sid, rid, -1)
    bmax = jnp.max(m, axis=0, keepdims=True)
    sel = (m == bmax).astype(jnp.bfloat16)
    upd = lax.dot_general(sel, h_new.astype(jnp.bfloat16),
                          (((0,), (0,)), ((), ())),
                          preferred_element_type=jnp.float32)
    hit = (bmax >= 0).T

    @pl.when(i == 0)
    def _():
        ns_ref[...] = state_ref[...]

    ns_ref[...] = jnp.where(hit[:STATES], upd[:STATES], ns_ref[...])


def _tc_gru(inputs, h_prev, state, W, U, b, W_out, b_out):
    out, new_state = pl.pallas_call(
        _gru_body,
        grid=(NBLK,),
        in_specs=[
            pl.BlockSpec((BLK, D_IN), lambda i: (i, 0)),
            pl.BlockSpec((BLK, DP), lambda i: (i, 0)),
            pl.BlockSpec((STATES, UNITS), lambda i: (0, 0)),
            pl.BlockSpec((D_IN, 3 * UNITS), lambda i: (0, 0)),
            pl.BlockSpec((UNITS, 3 * UNITS), lambda i: (0, 0)),
            pl.BlockSpec((1, 3 * UNITS), lambda i: (0, 0)),
            pl.BlockSpec((UNITS, 1), lambda i: (0, 0)),
            pl.BlockSpec((1, 1), lambda i: (0, 0)),
        ],
        out_specs=[
            pl.BlockSpec((BLK, 1), lambda i: (i, 0)),
            pl.BlockSpec((STATES, UNITS), lambda i: (0, 0)),
        ],
        out_shape=[
            jax.ShapeDtypeStruct((B, 1), jnp.float32),
            jax.ShapeDtypeStruct((STATES, UNITS), jnp.float32),
        ],
    )(inputs, h_prev, state, W, U, b.reshape(1, -1), W_out,
      b_out.reshape(1, 1))
    return out, new_state


def kernel(inputs, state, W, U, b, W_out, b_out):
    ids = jnp.clip(inputs[:, 0].astype(jnp.int32), 0, state.shape[0] - 1)
    nw = 32
    state_pad = jnp.pad(state, ((0, 0), (0, DP - UNITS)))
    idx3d = ids.reshape(nw, B // (nw * IDX_CH), IDX_CH)
    h_prev = _sc_gather_rows(state_pad, idx3d)
    return _tc_gru(inputs, h_prev, state, W, U, b, W_out, b_out)


# speedup vs baseline: 1.4579x; 1.0786x over previous
import functools

import jax
import jax.numpy as jnp
from jax import lax
from jax.experimental import pallas as pl
from jax.experimental.pallas import tpu as pltpu
from jax.experimental.pallas import tpu_sc as plsc

B = 16384
D_IN = 65
UNITS = 48
STATES = 1000
SPAD = 1024
BLK = 1024
NBLK = B // BLK


DP = 128
IDX_CH = 128


def _sc_gather_rows(table, idx3d):
    V, D = table.shape
    nw, nch, _ = idx3d.shape
    n = nw * nch * IDX_CH
    b_per_w = nch * IDX_CH
    mesh = plsc.VectorSubcoreMesh(core_axis_name="c", subcore_axis_name="s")

    @functools.partial(
        pl.kernel, mesh=mesh,
        out_type=jax.ShapeDtypeStruct((n, D), jnp.float32),
        scratch_types=[
            pltpu.VMEM((nch, IDX_CH), jnp.int32),
            pltpu.VMEM((b_per_w, D), jnp.float32),
            pltpu.SemaphoreType.DMA,
        ],
    )
    def k(table_hbm, idx_hbm, out_hbm, idx_v, rows_v, sem):
        wid = lax.axis_index("s") * 2 + lax.axis_index("c")
        base = wid * b_per_w
        pltpu.sync_copy(idx_hbm.at[wid], idx_v)
        cps = [
            pltpu.async_copy(table_hbm.at[idx_v.at[j]],
                             rows_v.at[pl.ds(j * IDX_CH, IDX_CH)], sem)
            for j in range(nch)
        ]
        for cp in cps:
            cp.wait()
        pltpu.sync_copy(rows_v, out_hbm.at[pl.ds(base, b_per_w)])

    return k(table, idx3d)


def _gru_body(x_ref, h_ref, state_ref, W_ref, U_ref, b_ref, Wo_ref, bo_ref,
              out_ref, ns_ref):
    i = pl.program_id(0)
    x = x_ref[...]
    h = h_ref[:, :UNITS]
    xp = jnp.dot(x, W_ref[...], preferred_element_type=jnp.float32) + b_ref[...]
    hp = jnp.dot(h, U_ref[...], preferred_element_type=jnp.float32)
    u = UNITS
    z = jax.nn.sigmoid(xp[:, :u] + hp[:, :u])
    r = jax.nn.sigmoid(xp[:, u:2 * u] + hp[:, u:2 * u])
    hh = jnp.tanh(xp[:, 2 * u:] + r * hp[:, 2 * u:])
    h_new = z * h + (1.0 - z) * hh
    head = jax.nn.sigmoid(
        jnp.dot(h_new, Wo_ref[...], preferred_element_type=jnp.float32)
        + bo_ref[...])
    out_ref[...] = head[:, 0]

    ids = x[:, 0:1].astype(jnp.int32)
    sid = lax.broadcasted_iota(jnp.int32, (BLK, SPAD), 1)
    rid = lax.broadcasted_iota(jnp.int32, (BLK, SPAD), 0)
    m = jnp.where(ids == sid, rid, -1)
    bmax = jnp.max(m, axis=0, keepdims=True)
    sel = (m == bmax).astype(jnp.bfloat16)
    hsrc = jnp.concatenate(
        [h_new.astype(jnp.bfloat16),
         jnp.ones((BLK, 1), jnp.bfloat16)], axis=1)
    res = lax.dot_general(sel, hsrc, (((0,), (0,)), ((), ())),
                          preferred_element_type=jnp.float32)
    hit = res[:, UNITS:UNITS + 1] == 1.0

    @pl.when(i == 0)
    def _():
        ns_ref[...] = state_ref[...]

    ns_ref[...] = jnp.where(hit[:STATES], res[:STATES, :UNITS], ns_ref[...])


def _tc_gru(inputs, h_prev, state, W, U, b, W_out, b_out):
    out, new_state = pl.pallas_call(
        _gru_body,
        grid=(NBLK,),
        in_specs=[
            pl.BlockSpec((BLK, D_IN), lambda i: (i, 0)),
            pl.BlockSpec((BLK, DP), lambda i: (i, 0)),
            pl.BlockSpec((STATES, UNITS), lambda i: (0, 0)),
            pl.BlockSpec((D_IN, 3 * UNITS), lambda i: (0, 0)),
            pl.BlockSpec((UNITS, 3 * UNITS), lambda i: (0, 0)),
            pl.BlockSpec((1, 3 * UNITS), lambda i: (0, 0)),
            pl.BlockSpec((UNITS, 1), lambda i: (0, 0)),
            pl.BlockSpec((1, 1), lambda i: (0, 0)),
        ],
        out_specs=[
            pl.BlockSpec((BLK,), lambda i: (i,)),
            pl.BlockSpec((STATES, UNITS), lambda i: (0, 0)),
        ],
        out_shape=[
            jax.ShapeDtypeStruct((B,), jnp.float32),
            jax.ShapeDtypeStruct((STATES, UNITS), jnp.float32),
        ],
    )(inputs, h_prev, state, W, U, b.reshape(1, -1), W_out,
      b_out.reshape(1, 1))
    return out.reshape(B, 1), new_state


def kernel(inputs, state, W, U, b, W_out, b_out):
    ids = jnp.clip(inputs[:, 0].astype(jnp.int32), 0, state.shape[0] - 1)
    nw = 32
    state_pad = jnp.pad(state, ((0, 0), (0, DP - UNITS)))
    idx3d = ids.reshape(nw, B // (nw * IDX_CH), IDX_CH)
    h_prev = _sc_gather_rows(state_pad, idx3d)
    return _tc_gru(inputs, h_prev, state, W, U, b, W_out, b_out)


# speedup vs baseline: 1.7183x; 1.1786x over previous
import functools

import jax
import jax.numpy as jnp
from jax import lax
from jax.experimental import pallas as pl
from jax.experimental.pallas import tpu as pltpu
from jax.experimental.pallas import tpu_sc as plsc

B = 16384
D_IN = 65
UNITS = 48
STATES = 1000
SPAD = 1024
BLK = 1024
NBLK = B // BLK
DP = 128
IDX_CH = 128
NW = 32
BPW = B // NW
SLOTS_PW = SPAD // NW


def _sc_gather_and_tables(table, idx3d):
    V, D = table.shape
    nw, nch, _ = idx3d.shape
    mesh = plsc.VectorSubcoreMesh(core_axis_name="c", subcore_axis_name="s")

    @functools.partial(
        pl.kernel, mesh=mesh,
        compiler_params=pltpu.CompilerParams(needs_layout_passes=False),
        out_type=[
            jax.ShapeDtypeStruct((B, D), jnp.float32),
            jax.ShapeDtypeStruct((NW * SPAD,), jnp.int32),
        ],
        scratch_types=[
            pltpu.VMEM((nch, IDX_CH), jnp.int32),
            pltpu.VMEM((BPW, D), jnp.float32),
            pltpu.VMEM((SPAD,), jnp.int32),
            pltpu.VMEM((16,), jnp.int32),
            pltpu.SemaphoreType.DMA,
        ],
    )
    def k(table_hbm, idx_hbm, out_hbm, wt_hbm, idx_v, rows_v, tab_v, scr_v,
          sem):
        wid = lax.axis_index("s") * 2 + lax.axis_index("c")
        base = wid * BPW
        pltpu.sync_copy(idx_hbm.at[wid], idx_v)
        cps = [
            pltpu.async_copy(table_hbm.at[idx_v.at[j]],
                             rows_v.at[pl.ds(j * IDX_CH, IDX_CH)], sem)
            for j in range(nch)
        ]
        neg = jnp.full((16,), -1, jnp.int32)
        for t in range(SPAD // 16):
            tab_v[pl.ds(t * 16, 16)] = neg
        lane = lax.iota(jnp.int32, 16)
        shifts = [jnp.minimum(lane + k, 15) for k in range(1, 16)]
        guards = [lane <= 15 - k for k in range(1, 16)]
        for j in range(nch):
            for t in range(IDX_CH // 16):
                ids16 = idx_v[j, pl.ds(t * 16, 16)]
                bidx = lane + (base + j * IDX_CH + t * 16)
                scr_v[...] = ids16
                not_last = lane < 0
                for k in range(15):
                    g = plsc.load_gather(scr_v, [shifts[k]])
                    not_last = not_last | ((ids16 == g) & guards[k])
                plsc.store_scatter(tab_v, [ids16], bidx,
                                   mask=jnp.logical_not(not_last))
        pltpu.sync_copy(tab_v, wt_hbm.at[pl.ds(wid * SPAD, SPAD)])
        for cp in cps:
            cp.wait()
        pltpu.sync_copy(rows_v, out_hbm.at[pl.ds(base, BPW)])

    return k(table, idx3d)


SLOTS_B = 128
NW_B = SPAD // SLOTS_B


def _sc_resolve_state(wt, h_all):
    mesh = plsc.VectorSubcoreMesh(core_axis_name="c", subcore_axis_name="s")

    @functools.partial(
        pl.kernel, mesh=mesh,
        compiler_params=pltpu.CompilerParams(needs_layout_passes=False),
        out_type=jax.ShapeDtypeStruct((SPAD, DP), jnp.float32),
        scratch_types=[
            pltpu.VMEM((NW, SLOTS_B), jnp.int32),
            pltpu.VMEM((SLOTS_B,), jnp.int32),
            pltpu.VMEM((SLOTS_B, DP), jnp.float32),
            pltpu.SemaphoreType.DMA,
            pltpu.SemaphoreType.DMA,
        ],
    )
    def k(wt_hbm, hall_hbm, out_hbm, mt_v, fi_v, rows_v, sem, gsem):
        wid = lax.axis_index("s") * 2 + lax.axis_index("c")

        @pl.when(wid < NW_B)
        def _():
            s0 = wid * SLOTS_B
            cps = [
                pltpu.async_copy(
                    wt_hbm.at[pl.ds(w * SPAD + s0, SLOTS_B)],
                    mt_v.at[w], sem)
                for w in range(NW)
            ]
            for cp in cps:
                cp.wait()
            for g in range(SLOTS_B // 16):
                acc = mt_v[0, pl.ds(g * 16, 16)]
                for j in range(1, NW):
                    acc = jnp.maximum(acc, mt_v[j, pl.ds(g * 16, 16)])
                slot = lax.iota(jnp.int32, 16) + (B + s0 + g * 16)
                fi_v[pl.ds(g * 16, 16)] = jnp.where(acc >= 0, acc, slot)
            pltpu.async_copy(hall_hbm.at[fi_v], rows_v, gsem).wait()
            pltpu.sync_copy(rows_v, out_hbm.at[pl.ds(s0, SLOTS_B)])

    return k(wt, h_all)


def _gru_body(x_ref, h_ref, state_ref, W_ref, U_ref, b_ref, Wo_ref, bo_ref,
              out_ref, hall_ref):
    i = pl.program_id(0)

    @pl.when(i < NBLK)
    def _():
        x = x_ref[...]
        h = h_ref[:, :UNITS]
        xp = (jnp.dot(x, W_ref[...], preferred_element_type=jnp.float32)
              + b_ref[...])
        hp = jnp.dot(h, U_ref[...], preferred_element_type=jnp.float32)
        u = UNITS
        z = jax.nn.sigmoid(xp[:, :u] + hp[:, :u])
        r = jax.nn.sigmoid(xp[:, u:2 * u] + hp[:, u:2 * u])
        hh = jnp.tanh(xp[:, 2 * u:] + r * hp[:, 2 * u:])
        h_new = z * h + (1.0 - z) * hh
        head = jax.nn.sigmoid(
            jnp.dot(h_new, Wo_ref[...], preferred_element_type=jnp.float32)
            + bo_ref[...])
        out_ref[...] = head[:, 0]
        hall_ref[:, :UNITS] = h_new

    @pl.when(i == NBLK)
    def _():
        hall_ref[...] = state_ref[...]


def _tc_gru(inputs, h_prev, state_pad, W, U, b, W_out, b_out):
    cap = lambda i: jnp.minimum(i, NBLK - 1)
    out, h_all = pl.pallas_call(
        _gru_body,
        grid=(NBLK + 1,),
        in_specs=[
            pl.BlockSpec((BLK, D_IN), lambda i: (cap(i), 0)),
            pl.BlockSpec((BLK, DP), lambda i: (cap(i), 0)),
            pl.BlockSpec((SPAD, DP), lambda i: (0, 0)),
            pl.BlockSpec((D_IN, 3 * UNITS), lambda i: (0, 0)),
            pl.BlockSpec((UNITS, 3 * UNITS), lambda i: (0, 0)),
            pl.BlockSpec((1, 3 * UNITS), lambda i: (0, 0)),
            pl.BlockSpec((UNITS, 1), lambda i: (0, 0)),
            pl.BlockSpec((1, 1), lambda i: (0, 0)),
        ],
        out_specs=[
            pl.BlockSpec((BLK,), lambda i: (cap(i),)),
            pl.BlockSpec((BLK, DP), lambda i: (i, 0)),
        ],
        out_shape=[
            jax.ShapeDtypeStruct((B,), jnp.float32),
            jax.ShapeDtypeStruct((B + SPAD, DP), jnp.float32),
        ],
    )(inputs, h_prev, state_pad, W, U, b.reshape(1, -1), W_out,
      b_out.reshape(1, 1))
    return out, h_all


def kernel(inputs, state, W, U, b, W_out, b_out):
    ids = jnp.clip(inputs[:, 0].astype(jnp.int32), 0, state.shape[0] - 1)
    state_pad = jnp.pad(state, ((0, SPAD - STATES), (0, DP - UNITS)))
    idx3d = ids.reshape(NW, BPW // IDX_CH, IDX_CH)
    h_prev, wt = _sc_gather_and_tables(state_pad, idx3d)
    out, h_all = _tc_gru(inputs, h_prev, state_pad, W, U, b, W_out, b_out)
    ns_pad = _sc_resolve_state(wt, h_all)
    return out.reshape(B, 1), ns_pad[:STATES, :UNITS]


# speedup vs baseline: 1.8477x; 1.0754x over previous
import functools

import jax
import jax.numpy as jnp
from jax import lax
from jax.experimental import pallas as pl
from jax.experimental.pallas import tpu as pltpu
from jax.experimental.pallas import tpu_sc as plsc

B = 16384
D_IN = 65
UNITS = 48
STATES = 1000
SPAD = 1024
BLK = 1024
NBLK = B // BLK
DP = 128
IDX_CH = 128
NW = 32
BPW = B // NW
SLOTS_PW = SPAD // NW


def _sc_gather_and_tables(table, idx3d):
    V, D = table.shape
    nw, nch, _ = idx3d.shape
    mesh = plsc.VectorSubcoreMesh(core_axis_name="c", subcore_axis_name="s")

    @functools.partial(
        pl.kernel, mesh=mesh,
        compiler_params=pltpu.CompilerParams(needs_layout_passes=False),
        out_type=[
            jax.ShapeDtypeStruct((B, D), jnp.float32),
            jax.ShapeDtypeStruct((NW * SPAD,), jnp.int32),
        ],
        scratch_types=[
            pltpu.VMEM((nch, IDX_CH), jnp.int32),
            pltpu.VMEM((BPW, D), jnp.float32),
            pltpu.VMEM((SPAD,), jnp.int32),
            pltpu.VMEM((16,), jnp.int32),
            pltpu.SemaphoreType.DMA,
        ],
    )
    def k(table_hbm, idx_hbm, out_hbm, wt_hbm, idx_v, rows_v, tab_v, scr_v,
          sem):
        wid = lax.axis_index("s") * 2 + lax.axis_index("c")
        base = wid * BPW
        pltpu.sync_copy(idx_hbm.at[wid], idx_v)
        cps = [
            pltpu.async_copy(table_hbm.at[idx_v.at[j]],
                             rows_v.at[pl.ds(j * IDX_CH, IDX_CH)], sem)
            for j in range(nch)
        ]
        neg = jnp.full((16,), -1, jnp.int32)
        for t in range(SPAD // 16):
            tab_v[pl.ds(t * 16, 16)] = neg
        lane = lax.iota(jnp.int32, 16)
        shifts = [jnp.minimum(lane + k, 15) for k in range(1, 16)]
        guards = [lane <= 15 - k for k in range(1, 16)]
        for j in range(nch):
            for t in range(IDX_CH // 16):
                ids16 = idx_v[j, pl.ds(t * 16, 16)]
                bidx = lane + (base + j * IDX_CH + t * 16)
                scr_v[...] = ids16
                not_last = lane < 0
                for k in range(15):
                    g = plsc.load_gather(scr_v, [shifts[k]])
                    not_last = not_last | ((ids16 == g) & guards[k])
                plsc.store_scatter(tab_v, [ids16], bidx,
                                   mask=jnp.logical_not(not_last))
        pltpu.sync_copy(tab_v, wt_hbm.at[pl.ds(wid * SPAD, SPAD)])
        for cp in cps:
            cp.wait()
        pltpu.sync_copy(rows_v, out_hbm.at[pl.ds(base, BPW)])

    return k(table, idx3d)


SLOTS_B = 128
NW_B = SPAD // SLOTS_B


def _sc_resolve_state(wt, h_all):
    mesh = plsc.VectorSubcoreMesh(core_axis_name="c", subcore_axis_name="s")

    @functools.partial(
        pl.kernel, mesh=mesh,
        compiler_params=pltpu.CompilerParams(needs_layout_passes=False),
        out_type=jax.ShapeDtypeStruct((SPAD, DP), jnp.float32),
        scratch_types=[
            pltpu.VMEM((NW, SLOTS_B), jnp.int32),
            pltpu.VMEM((SLOTS_B,), jnp.int32),
            pltpu.VMEM((SLOTS_B, DP), jnp.float32),
            pltpu.SemaphoreType.DMA,
            pltpu.SemaphoreType.DMA,
        ],
    )
    def k(wt_hbm, hall_hbm, out_hbm, mt_v, fi_v, rows_v, sem, gsem):
        wid = lax.axis_index("s") * 2 + lax.axis_index("c")

        @pl.when(wid < NW_B)
        def _():
            s0 = wid * SLOTS_B
            cps = [
                pltpu.async_copy(
                    wt_hbm.at[pl.ds(w * SPAD + s0, SLOTS_B)],
                    mt_v.at[w], sem)
                for w in range(NW)
            ]
            for cp in cps:
                cp.wait()
            for g in range(SLOTS_B // 16):
                acc = mt_v[0, pl.ds(g * 16, 16)]
                for j in range(1, NW):
                    acc = jnp.maximum(acc, mt_v[j, pl.ds(g * 16, 16)])
                slot = lax.iota(jnp.int32, 16) + (B + s0 + g * 16)
                fi_v[pl.ds(g * 16, 16)] = jnp.where(acc >= 0, acc, slot)
            pltpu.async_copy(hall_hbm.at[fi_v], rows_v, gsem).wait()
            pltpu.sync_copy(rows_v, out_hbm.at[pl.ds(s0, SLOTS_B)])

    return k(wt, h_all)


def _gru_body(x_ref, h_ref, state_ref, Wz_ref, Wr_ref, Wh_ref,
              Uz_ref, Ur_ref, Uh_ref, bz_ref, br_ref, bh_ref,
              wo_ref, bo_ref, out_ref, hall_ref):
    i = pl.program_id(0)

    @pl.when(i < NBLK)
    def _():
        x = x_ref[...]
        h = h_ref[:, :UNITS]
        dot = lambda a, b: jnp.dot(a, b, preferred_element_type=jnp.float32)
        z = jax.nn.sigmoid(dot(x, Wz_ref[...]) + dot(h, Uz_ref[...])
                           + bz_ref[...])
        r = jax.nn.sigmoid(dot(x, Wr_ref[...]) + dot(h, Ur_ref[...])
                           + br_ref[...])
        hh = jnp.tanh(dot(x, Wh_ref[...]) + r * dot(h, Uh_ref[...])
                      + bh_ref[...])
        h_new = z * h + (1.0 - z) * hh
        head = jax.nn.sigmoid(
            lax.dot_general(wo_ref[...], h_new, (((1,), (1,)), ((), ())),
                            preferred_element_type=jnp.float32)
            + bo_ref[...])
        out_ref[...] = head
        hall_ref[:, :UNITS] = h_new

    @pl.when(i == NBLK)
    def _():
        hall_ref[...] = state_ref[...]


def _tc_gru(inputs, h_prev, state_pad, W, U, b, W_out, b_out):
    cap = lambda i: jnp.minimum(i, NBLK - 1)
    out, h_all = pl.pallas_call(
        _gru_body,
        grid=(NBLK + 1,),
        in_specs=[
            pl.BlockSpec((BLK, D_IN), lambda i: (cap(i), 0)),
            pl.BlockSpec((BLK, DP), lambda i: (cap(i), 0)),
            pl.BlockSpec((SPAD, DP), lambda i: (0, 0)),
            pl.BlockSpec((D_IN, UNITS), lambda i: (0, 0)),
            pl.BlockSpec((D_IN, UNITS), lambda i: (0, 0)),
            pl.BlockSpec((D_IN, UNITS), lambda i: (0, 0)),
            pl.BlockSpec((UNITS, UNITS), lambda i: (0, 0)),
            pl.BlockSpec((UNITS, UNITS), lambda i: (0, 0)),
            pl.BlockSpec((UNITS, UNITS), lambda i: (0, 0)),
            pl.BlockSpec((1, UNITS), lambda i: (0, 0)),
            pl.BlockSpec((1, UNITS), lambda i: (0, 0)),
            pl.BlockSpec((1, UNITS), lambda i: (0, 0)),
            pl.BlockSpec((1, UNITS), lambda i: (0, 0)),
            pl.BlockSpec((1, 1), lambda i: (0, 0)),
        ],
        out_specs=[
            pl.BlockSpec((1, BLK), lambda i: (0, cap(i))),
            pl.BlockSpec((BLK, DP), lambda i: (i, 0)),
        ],
        out_shape=[
            jax.ShapeDtypeStruct((1, B), jnp.float32),
            jax.ShapeDtypeStruct((B + SPAD, DP), jnp.float32),
        ],
    )(inputs, h_prev, state_pad,
      W[:, :UNITS], W[:, UNITS:2 * UNITS], W[:, 2 * UNITS:],
      U[:, :UNITS], U[:, UNITS:2 * UNITS], U[:, 2 * UNITS:],
      b[:UNITS].reshape(1, -1), b[UNITS:2 * UNITS].reshape(1, -1),
      b[2 * UNITS:].reshape(1, -1),
      W_out.reshape(1, UNITS), b_out.reshape(1, 1))
    return out, h_all


def kernel(inputs, state, W, U, b, W_out, b_out):
    ids = jnp.clip(inputs[:, 0].astype(jnp.int32), 0, state.shape[0] - 1)
    state_pad = jnp.pad(state, ((0, SPAD - STATES), (0, DP - UNITS)))
    idx3d = ids.reshape(NW, BPW // IDX_CH, IDX_CH)
    h_prev, wt = _sc_gather_and_tables(state_pad, idx3d)
    out2, h_all = _tc_gru(inputs, h_prev, state_pad, W, U, b, W_out, b_out)
    ns_pad = _sc_resolve_state(wt, h_all)
    return out2.reshape(B, 1), ns_pad[:STATES, :UNITS]


# speedup vs baseline: 1.9782x; 1.0706x over previous
import functools

import jax
import jax.numpy as jnp
from jax import lax
from jax.experimental import pallas as pl
from jax.experimental.pallas import tpu as pltpu
from jax.experimental.pallas import tpu_sc as plsc

B = 16384
D_IN = 65
UNITS = 48
STATES = 1000
SPAD = 1024
BLK = 2048
NBLK = B // BLK
DP = 128
IDX_CH = 128
NW = 32
BPW = B // NW
SLOTS_PW = SPAD // NW


def _sc_gather_and_tables(table, idx3d):
    V, D = table.shape
    nw, nch, _ = idx3d.shape
    mesh = plsc.VectorSubcoreMesh(core_axis_name="c", subcore_axis_name="s")

    @functools.partial(
        pl.kernel, mesh=mesh,
        compiler_params=pltpu.CompilerParams(needs_layout_passes=False),
        out_type=[
            jax.ShapeDtypeStruct((B, D), jnp.float32),
            jax.ShapeDtypeStruct((NW * SPAD,), jnp.int32),
        ],
        scratch_types=[
            pltpu.VMEM((nch, IDX_CH), jnp.int32),
            pltpu.VMEM((BPW, D), jnp.float32),
            pltpu.VMEM((SPAD,), jnp.int32),
            pltpu.VMEM((16,), jnp.int32),
            pltpu.SemaphoreType.DMA,
        ],
    )
    def k(table_hbm, idx_hbm, out_hbm, wt_hbm, idx_v, rows_v, tab_v, scr_v,
          sem):
        wid = lax.axis_index("s") * 2 + lax.axis_index("c")
        base = wid * BPW
        pltpu.sync_copy(idx_hbm.at[wid], idx_v)
        cps = [
            pltpu.async_copy(table_hbm.at[idx_v.at[j]],
                             rows_v.at[pl.ds(j * IDX_CH, IDX_CH)], sem)
            for j in range(nch)
        ]
        neg = jnp.full((16,), -1, jnp.int32)
        for t in range(SPAD // 16):
            tab_v[pl.ds(t * 16, 16)] = neg
        lane = lax.iota(jnp.int32, 16)
        shifts = [jnp.minimum(lane + k, 15) for k in range(1, 16)]
        guards = [lane <= 15 - k for k in range(1, 16)]
        for j in range(nch):
            for t in range(IDX_CH // 16):
                ids16 = idx_v[j, pl.ds(t * 16, 16)]
                bidx = lane + (base + j * IDX_CH + t * 16)
                scr_v[...] = ids16
                not_last = lane < 0
                for k in range(15):
                    g = plsc.load_gather(scr_v, [shifts[k]])
                    not_last = not_last | ((ids16 == g) & guards[k])
                plsc.store_scatter(tab_v, [ids16], bidx,
                                   mask=jnp.logical_not(not_last))
        pltpu.sync_copy(tab_v, wt_hbm.at[pl.ds(wid * SPAD, SPAD)])
        for cp in cps:
            cp.wait()
        pltpu.sync_copy(rows_v, out_hbm.at[pl.ds(base, BPW)])

    return k(table, idx3d)


SLOTS_B = 128
NW_B = SPAD // SLOTS_B


def _sc_resolve_state(wt, h_all):
    mesh = plsc.VectorSubcoreMesh(core_axis_name="c", subcore_axis_name="s")

    @functools.partial(
        pl.kernel, mesh=mesh,
        compiler_params=pltpu.CompilerParams(needs_layout_passes=False),
        out_type=jax.ShapeDtypeStruct((SPAD, DP), jnp.float32),
        scratch_types=[
            pltpu.VMEM((NW, SLOTS_B), jnp.int32),
            pltpu.VMEM((SLOTS_B,), jnp.int32),
            pltpu.VMEM((SLOTS_B, DP), jnp.float32),
            pltpu.SemaphoreType.DMA,
            pltpu.SemaphoreType.DMA,
        ],
    )
    def k(wt_hbm, hall_hbm, out_hbm, mt_v, fi_v, rows_v, sem, gsem):
        wid = lax.axis_index("s") * 2 + lax.axis_index("c")

        @pl.when(wid < NW_B)
        def _():
            s0 = wid * SLOTS_B
            cps = [
                pltpu.async_copy(
                    wt_hbm.at[pl.ds(w * SPAD + s0, SLOTS_B)],
                    mt_v.at[w], sem)
                for w in range(NW)
            ]
            for cp in cps:
                cp.wait()
            for g in range(SLOTS_B // 16):
                acc = mt_v[0, pl.ds(g * 16, 16)]
                for j in range(1, NW):
                    acc = jnp.maximum(acc, mt_v[j, pl.ds(g * 16, 16)])
                slot = lax.iota(jnp.int32, 16) + (B + s0 + g * 16)
                fi_v[pl.ds(g * 16, 16)] = jnp.where(acc >= 0, acc, slot)
            pltpu.async_copy(hall_hbm.at[fi_v], rows_v, gsem).wait()
            pltpu.sync_copy(rows_v, out_hbm.at[pl.ds(s0, SLOTS_B)])

    return k(wt, h_all)


def _gru_body(x_ref, h_ref, state_ref, Wz_ref, Wr_ref, Wh_ref,
              Uz_ref, Ur_ref, Uh_ref, bz_ref, br_ref, bh_ref,
              wo_ref, bo_ref, out_ref, hall_ref):
    i = pl.program_id(0)

    @pl.when(i < NBLK)
    def _():
        x = x_ref[...]
        h = h_ref[:, :UNITS]
        dot = lambda a, b: jnp.dot(a, b, preferred_element_type=jnp.float32)
        z = jax.nn.sigmoid(dot(x, Wz_ref[...]) + dot(h, Uz_ref[...])
                           + bz_ref[...])
        r = jax.nn.sigmoid(dot(x, Wr_ref[...]) + dot(h, Ur_ref[...])
                           + br_ref[...])
        hh = jnp.tanh(dot(x, Wh_ref[...]) + r * dot(h, Uh_ref[...])
                      + bh_ref[...])
        h_new = z * h + (1.0 - z) * hh
        head = jax.nn.sigmoid(
            lax.dot_general(wo_ref[...], h_new, (((1,), (1,)), ((), ())),
                            preferred_element_type=jnp.float32)
            + bo_ref[...])
        out_ref[...] = head
        hall_ref[:, :UNITS] = h_new

    @pl.when(i == NBLK)
    def _():
        hall_ref[:SPAD, :] = state_ref[...]


def _tc_gru(inputs, h_prev, state_pad, W, U, b, W_out, b_out):
    cap = lambda i: jnp.minimum(i, NBLK - 1)
    out, h_all = pl.pallas_call(
        _gru_body,
        grid=(NBLK + 1,),
        in_specs=[
            pl.BlockSpec((BLK, D_IN), lambda i: (cap(i), 0)),
            pl.BlockSpec((BLK, DP), lambda i: (cap(i), 0)),
            pl.BlockSpec((SPAD, DP), lambda i: (0, 0)),
            pl.BlockSpec((D_IN, UNITS), lambda i: (0, 0)),
            pl.BlockSpec((D_IN, UNITS), lambda i: (0, 0)),
            pl.BlockSpec((D_IN, UNITS), lambda i: (0, 0)),
            pl.BlockSpec((UNITS, UNITS), lambda i: (0, 0)),
            pl.BlockSpec((UNITS, UNITS), lambda i: (0, 0)),
            pl.BlockSpec((UNITS, UNITS), lambda i: (0, 0)),
            pl.BlockSpec((1, UNITS), lambda i: (0, 0)),
            pl.BlockSpec((1, UNITS), lambda i: (0, 0)),
            pl.BlockSpec((1, UNITS), lambda i: (0, 0)),
            pl.BlockSpec((1, UNITS), lambda i: (0, 0)),
            pl.BlockSpec((1, 1), lambda i: (0, 0)),
        ],
        out_specs=[
            pl.BlockSpec((1, BLK), lambda i: (0, cap(i))),
            pl.BlockSpec((BLK, DP), lambda i: (i, 0)),
        ],
        out_shape=[
            jax.ShapeDtypeStruct((1, B), jnp.float32),
            jax.ShapeDtypeStruct((B + BLK, DP), jnp.float32),
        ],
    )(inputs, h_prev, state_pad,
      W[:, :UNITS], W[:, UNITS:2 * UNITS], W[:, 2 * UNITS:],
      U[:, :UNITS], U[:, UNITS:2 * UNITS], U[:, 2 * UNITS:],
      b[:UNITS].reshape(1, -1), b[UNITS:2 * UNITS].reshape(1, -1),
      b[2 * UNITS:].reshape(1, -1),
      W_out.reshape(1, UNITS), b_out.reshape(1, 1))
    return out, h_all


def kernel(inputs, state, W, U, b, W_out, b_out):
    ids = jnp.clip(inputs[:, 0].astype(jnp.int32), 0, state.shape[0] - 1)
    state_pad = jnp.pad(state, ((0, SPAD - STATES), (0, DP - UNITS)))
    idx3d = ids.reshape(NW, BPW // IDX_CH, IDX_CH)
    h_prev, wt = _sc_gather_and_tables(state_pad, idx3d)
    out2, h_all = _tc_gru(inputs, h_prev, state_pad, W, U, b, W_out, b_out)
    ns_pad = _sc_resolve_state(wt, h_all)
    return out2.reshape(B, 1), ns_pad[:STATES, :UNITS]
